# baseline (speedup 1.0000x reference)
import functools

import jax
import jax.numpy as jnp
from jax import lax
from jax.experimental import pallas as pl
from jax.experimental.pallas import tpu as pltpu
from jax.experimental.pallas import tpu_sc as plsc

N = 10000
D = 256
H = 128
E = 160000

NC = 2
NS = 16

CH = 128
NCHUNK = E // CH
CPT_A = 80
CPT_B = NCHUNK - 15 * CPT_A
PH = 40

EPT_D = E // (NC * NS)
NFULL_D = EPT_D // CH
TAIL_D = EPT_D - NFULL_D * CH

ROWS_A = 632
ROWS_B = N - 15 * ROWS_A

_mesh = plsc.VectorSubcoreMesh(core_axis_name="c", subcore_axis_name="s")


@functools.partial(
    pl.kernel,
    out_type=jax.ShapeDtypeStruct((NC, N), jnp.float32),
    mesh=_mesh,
    scratch_types=[
        pltpu.VMEM_SHARED((N,), jnp.float32),
        pltpu.VMEM((CH,), jnp.float32),
        pltpu.VMEM((CH,), jnp.int32),
        pltpu.VMEM((TAIL_D,), jnp.int32),
    ],
)
def _deg_kernel(dst, zeros1, out, accd, ones_b, didx, didx_t):
    c = lax.axis_index("c")
    s = lax.axis_index("s")

    @pl.when(s == 0)
    def _():
        pltpu.sync_copy(zeros1, accd)

    for j in range(CH // 16):
        ones_b[pl.ds(j * 16, 16)] = jnp.full((16,), 1.0, dtype=jnp.float32)
    plsc.subcore_barrier()

    base = (c * NS + s) * EPT_D

    def body(i, _):
        b = pl.multiple_of(base + i * CH, 8)
        pltpu.sync_copy(dst.at[pl.ds(b, CH)], didx)
        pltpu.sync_copy(ones_b, accd.at[didx], add=True)
        return ()

    lax.fori_loop(0, NFULL_D, body, ())
    bt = pl.multiple_of(base + NFULL_D * CH, 8)
    pltpu.sync_copy(dst.at[pl.ds(bt, TAIL_D)], didx_t)
    pltpu.sync_copy(ones_b.at[pl.ds(0, TAIL_D)], accd.at[didx_t], add=True)

    plsc.subcore_barrier()

    @pl.when(s == 0)
    def _():
        pltpu.sync_copy(accd, out.at[c])


@functools.partial(
    pl.kernel,
    out_type=jax.ShapeDtypeStruct((NC, N, H), jnp.float32),
    mesh=_mesh,
    scratch_types=[
        pltpu.VMEM_SHARED((N, H), jnp.float32),
        pltpu.VMEM((PH, CH), jnp.int32),
        pltpu.VMEM((PH, CH), jnp.int32),
        pltpu.VMEM((2, CH, H), jnp.float32),
        pltpu.SemaphoreType.DMA,
        pltpu.SemaphoreType.DMA,
        pltpu.SemaphoreType.DMA,
        pltpu.SemaphoreType.DMA,
    ],
)
def _agg_kernel(hs, src2d, dst2d, zeros2, out, acc, sbuf, dbuf, gbuf,
                sem0, sem1, sem2, sem3):
    c = lax.axis_index("c")
    s = lax.axis_index("s")
    r0 = pl.multiple_of(s * ROWS_A, 8)

    def gather(i, slot, sem):
        return pltpu.make_async_copy(hs.at[c].at[sbuf.at[i]],
                                     gbuf.at[slot], sem)

    def stage0():
        cs = pl.multiple_of(jnp.where(s < 15, s * CPT_A, 15 * CPT_A), 8)
        return (pltpu.make_async_copy(src2d.at[pl.ds(cs, PH)], sbuf, sem0),
                pltpu.make_async_copy(dst2d.at[pl.ds(cs, PH)], dbuf, sem1))

    for cp in stage0():
        cp.start()

    @pl.when(s < 15)
    def _():
        pltpu.sync_copy(zeros2.at[pl.ds(r0, ROWS_A)],
                        acc.at[pl.ds(r0, ROWS_A)])

    @pl.when(s == 15)
    def _():
        pltpu.sync_copy(zeros2.at[pl.ds(15 * ROWS_A, ROWS_B)],
                        acc.at[pl.ds(15 * ROWS_A, ROWS_B)])

    plsc.subcore_barrier()
    for cp in stage0():
        cp.wait()

    for p in range(2):
        if p == 0:
            nch = PH
        else:
            @pl.when(s < 15)
            def _():
                cs = pl.multiple_of(s * CPT_A + PH, 8)
                pltpu.sync_copy(src2d.at[pl.ds(cs, PH)], sbuf)
                pltpu.sync_copy(dst2d.at[pl.ds(cs, PH)], dbuf)

            @pl.when(s == 15)
            def _():
                cs = 15 * CPT_A + PH
                nlast = CPT_B - PH
                pltpu.sync_copy(src2d.at[pl.ds(cs, nlast)],
                                sbuf.at[pl.ds(0, nlast)])
                pltpu.sync_copy(dst2d.at[pl.ds(cs, nlast)],
                                dbuf.at[pl.ds(0, nlast)])

            nch = jnp.where(s < 15, PH, CPT_B - PH)

        def scat(i, slot, sem):
            return pltpu.make_async_copy(gbuf.at[slot], acc.at[dbuf.at[i]],
                                         sem)

        gather(0, 0, sem0).start()
        gather(1, 1, sem1).start()

        def body(i2, _):
            a = 2 * i2
            gather(a, 0, sem0).wait()
            pltpu.async_copy(gbuf.at[0], acc.at[dbuf.at[a]], sem2, add=True)
            gather(a + 1, 1, sem1).wait()
            pltpu.async_copy(gbuf.at[1], acc.at[dbuf.at[a + 1]], sem3,
                             add=True)

            @pl.when(a + 2 < nch)
            def _():
                scat(a, 0, sem2).wait()
                gather(a + 2, 0, sem0).start()
                scat(a + 1, 1, sem3).wait()
                gather(a + 3, 1, sem1).start()

            return ()

        lax.fori_loop(0, nch // 2, body, ())
        scat(nch - 2, 0, sem2).wait()
        scat(nch - 1, 1, sem3).wait()

    plsc.subcore_barrier()

    @pl.when(s < 15)
    def _():
        pltpu.sync_copy(acc.at[pl.ds(r0, ROWS_A)],
                        out.at[c].at[pl.ds(r0, ROWS_A)])

    @pl.when(s == 15)
    def _():
        pltpu.sync_copy(acc.at[pl.ds(15 * ROWS_A, ROWS_B)],
                        out.at[c].at[pl.ds(15 * ROWS_A, ROWS_B)])


RB = 400

def _dinv_block(dp):
    deg = dp[0, :, 0] + dp[1, :, 0] + 1.0
    return lax.rsqrt(deg)


def _mm1_body(x_ref, w_ref, o_ref):
    o_ref[...] = jnp.dot(x_ref[...], w_ref[...],
                         preferred_element_type=jnp.float32)


_mm1 = pl.pallas_call(
    _mm1_body,
    grid=(N // RB,),
    in_specs=[
        pl.BlockSpec((RB, D), lambda i: (i, 0)),
        pl.BlockSpec((D, D), lambda i: (0, 0)),
    ],
    out_specs=pl.BlockSpec((RB, D), lambda i: (i, 0)),
    out_shape=jax.ShapeDtypeStruct((N, D), jnp.float32),
)


def _scale_body(dp_ref, h_ref, o_ref):
    hs = h_ref[...] * _dinv_block(dp_ref[...])[:, None]
    o_ref[0] = hs[:, :H]
    o_ref[1] = hs[:, H:]


_scale = pl.pallas_call(
    _scale_body,
    grid=(N // RB,),
    in_specs=[
        pl.BlockSpec((NC, RB, 1), lambda i: (0, i, 0)),
        pl.BlockSpec((RB, D), lambda i: (i, 0)),
    ],
    out_specs=pl.BlockSpec((NC, RB, H), lambda i: (0, i, 0)),
    out_shape=jax.ShapeDtypeStruct((NC, N, H), jnp.float32),
)


def _mid_body(dp_ref, agg_ref, hs_ref, b_ref, w_ref, o_ref):
    dinv = _dinv_block(dp_ref[...])
    pre = jnp.concatenate(
        [agg_ref[0] + hs_ref[0], agg_ref[1] + hs_ref[1]], axis=1)
    t = jnp.maximum(pre * dinv[:, None] + b_ref[...], 0.0)
    h2 = jnp.dot(t, w_ref[...], preferred_element_type=jnp.float32)
    hs2 = h2 * dinv[:, None]
    o_ref[0] = hs2[:, :H]
    o_ref[1] = hs2[:, H:]


_mid = pl.pallas_call(
    _mid_body,
    grid=(N // RB,),
    in_specs=[
        pl.BlockSpec((NC, RB, 1), lambda i: (0, i, 0)),
        pl.BlockSpec((NC, RB, H), lambda i: (0, i, 0)),
        pl.BlockSpec((NC, RB, H), lambda i: (0, i, 0)),
        pl.BlockSpec((1, D), lambda i: (0, 0)),
        pl.BlockSpec((D, D), lambda i: (0, 0)),
    ],
    out_specs=pl.BlockSpec((NC, RB, H), lambda i: (0, i, 0)),
    out_shape=jax.ShapeDtypeStruct((NC, N, H), jnp.float32),
)


def _fin_body(dp_ref, agg_ref, hs_ref, b_ref, o_ref):
    dinv = _dinv_block(dp_ref[...])
    pre = jnp.concatenate(
        [agg_ref[0] + hs_ref[0], agg_ref[1] + hs_ref[1]], axis=1)
    o_ref[...] = pre * dinv[:, None] + b_ref[...]


_fin = pl.pallas_call(
    _fin_body,
    grid=(N // RB,),
    in_specs=[
        pl.BlockSpec((NC, RB, 1), lambda i: (0, i, 0)),
        pl.BlockSpec((NC, RB, H), lambda i: (0, i, 0)),
        pl.BlockSpec((NC, RB, H), lambda i: (0, i, 0)),
        pl.BlockSpec((1, D), lambda i: (0, 0)),
    ],
    out_specs=pl.BlockSpec((RB, D), lambda i: (i, 0)),
    out_shape=jax.ShapeDtypeStruct((N, D), jnp.float32),
)


@jax.jit
def kernel(x, edge_idx, W1, b1, W2, b2):
    zeros1 = jnp.zeros((N,), jnp.float32)
    zeros2 = jnp.zeros((N, H), jnp.float32)
    src2d = edge_idx[0].reshape(NCHUNK, CH)
    dst = edge_idx[1]
    dst2d = dst.reshape(NCHUNK, CH)
    dp = _deg_kernel(dst, zeros1)
    dp3 = dp.reshape(NC, N, 1)
    h1 = _mm1(x, W1)
    hs1 = _scale(dp3, h1)
    agg1 = _agg_kernel(hs1, src2d, dst2d, zeros2)
    hs2 = _mid(dp3, agg1, hs1, b1.reshape(1, D), W2)
    agg2 = _agg_kernel(hs2, src2d, dst2d, zeros2)
    return _fin(dp3, agg2, hs2, b2.reshape(1, D))


# speedup vs baseline: 1.2401x; 1.2401x over previous
import functools

import jax
import jax.numpy as jnp
from jax import lax
from jax.experimental import pallas as pl
from jax.experimental.pallas import tpu as pltpu
from jax.experimental.pallas import tpu_sc as plsc

N = 10000
D = 256
H = 128
E = 160000

NC = 2
NS = 16

CH = 128
NCHUNK = E // CH
CPT_A = 80
CPT_B = NCHUNK - 15 * CPT_A
PH = 40

CPW_D = 40
CPW_D_LAST = 1250 - 31 * CPW_D

ROWS_A = 632
ROWS_B = N - 15 * ROWS_A

_mesh = plsc.VectorSubcoreMesh(core_axis_name="c", subcore_axis_name="s")


@functools.partial(
    pl.kernel,
    out_type=jax.ShapeDtypeStruct((NC, N), jnp.float32),
    mesh=_mesh,
    scratch_types=[
        pltpu.VMEM_SHARED((N,), jnp.float32),
        pltpu.VMEM((CH,), jnp.float32),
        pltpu.VMEM((CPW_D, CH), jnp.int32),
        pltpu.SemaphoreType.DMA,
    ],
)
def _deg_kernel(dst2d, zeros1, out, accd, ones_b, dbufd, semd):
    c = lax.axis_index("c")
    s = lax.axis_index("s")
    w = c * NS + s

    @pl.when(w < 31)
    def _():
        cs = pl.multiple_of(w * CPW_D, 8)
        pltpu.make_async_copy(dst2d.at[pl.ds(cs, CPW_D)], dbufd, semd).start()

    @pl.when(w == 31)
    def _():
        pltpu.make_async_copy(dst2d.at[pl.ds(31 * CPW_D, CPW_D_LAST)],
                              dbufd.at[pl.ds(0, CPW_D_LAST)], semd).start()

    @pl.when(s == 0)
    def _():
        pltpu.sync_copy(zeros1, accd)

    for j in range(CH // 16):
        ones_b[pl.ds(j * 16, 16)] = jnp.full((16,), 1.0, dtype=jnp.float32)

    plsc.subcore_barrier()

    @pl.when(w < 31)
    def _():
        cs = pl.multiple_of(w * CPW_D, 8)
        pltpu.make_async_copy(dst2d.at[pl.ds(cs, CPW_D)], dbufd, semd).wait()

    @pl.when(w == 31)
    def _():
        pltpu.make_async_copy(dst2d.at[pl.ds(31 * CPW_D, CPW_D_LAST)],
                              dbufd.at[pl.ds(0, CPW_D_LAST)], semd).wait()

    nw = jnp.where(w < 31, CPW_D, CPW_D_LAST)

    def fire(i, _):
        pltpu.async_copy(ones_b, accd.at[dbufd.at[i]], semd, add=True)
        return ()

    lax.fori_loop(0, nw, fire, ())

    def drain(i, _):
        pltpu.make_async_copy(ones_b, accd.at[dbufd.at[i]], semd).wait()
        return ()

    lax.fori_loop(0, nw, drain, ())

    plsc.subcore_barrier()

    @pl.when(s == 0)
    def _():
        pltpu.sync_copy(accd, out.at[c])


@functools.partial(
    pl.kernel,
    out_type=jax.ShapeDtypeStruct((NC, N, H), jnp.float32),
    mesh=_mesh,
    scratch_types=[
        pltpu.VMEM_SHARED((N, H), jnp.float32),
        pltpu.VMEM((PH, CH), jnp.int32),
        pltpu.VMEM((PH, CH), jnp.int32),
        pltpu.VMEM((2, CH, H), jnp.float32),
        pltpu.SemaphoreType.DMA,
        pltpu.SemaphoreType.DMA,
    ],
)
def _agg_kernel(hs, src2d, dst2d, zeros2, out, acc, sbuf, dbuf, gbuf,
                sem0, sem1):
    c = lax.axis_index("c")
    s = lax.axis_index("s")
    r0 = pl.multiple_of(s * ROWS_A, 8)

    def gather(i, slot, sem):
        return pltpu.make_async_copy(hs.at[c].at[sbuf.at[i]],
                                     gbuf.at[slot], sem)

    def stage0():
        cs = pl.multiple_of(jnp.where(s < 15, s * CPT_A, 15 * CPT_A), 8)
        return (pltpu.make_async_copy(src2d.at[pl.ds(cs, PH)], sbuf, sem0),
                pltpu.make_async_copy(dst2d.at[pl.ds(cs, PH)], dbuf, sem1))

    for cp in stage0():
        cp.start()

    @pl.when(s < 15)
    def _():
        pltpu.sync_copy(zeros2.at[pl.ds(r0, ROWS_A)],
                        acc.at[pl.ds(r0, ROWS_A)])

    @pl.when(s == 15)
    def _():
        pltpu.sync_copy(zeros2.at[pl.ds(15 * ROWS_A, ROWS_B)],
                        acc.at[pl.ds(15 * ROWS_A, ROWS_B)])

    plsc.subcore_barrier()
    for cp in stage0():
        cp.wait()

    for p in range(2):
        if p == 0:
            nch = PH
        else:
            @pl.when(s < 15)
            def _():
                cs = pl.multiple_of(s * CPT_A + PH, 8)
                pltpu.sync_copy(src2d.at[pl.ds(cs, PH)], sbuf)
                pltpu.sync_copy(dst2d.at[pl.ds(cs, PH)], dbuf)

            @pl.when(s == 15)
            def _():
                cs = 15 * CPT_A + PH
                nlast = CPT_B - PH
                pltpu.sync_copy(src2d.at[pl.ds(cs, nlast)],
                                sbuf.at[pl.ds(0, nlast)])
                pltpu.sync_copy(dst2d.at[pl.ds(cs, nlast)],
                                dbuf.at[pl.ds(0, nlast)])

            nch = jnp.where(s < 15, PH, CPT_B - PH)

        gather(0, 0, sem0).start()

        def body(i2, _):
            a = 2 * i2
            gather(a + 1, 1, sem1).start()
            gather(a, 0, sem0).wait()
            pltpu.sync_copy(gbuf.at[0], acc.at[dbuf.at[a]], add=True)

            @pl.when(a + 2 < nch)
            def _():
                gather(a + 2, 0, sem0).start()

            gather(a + 1, 1, sem1).wait()
            pltpu.sync_copy(gbuf.at[1], acc.at[dbuf.at[a + 1]], add=True)
            return ()

        lax.fori_loop(0, nch // 2, body, ())

    plsc.subcore_barrier()

    @pl.when(s < 15)
    def _():
        pltpu.sync_copy(acc.at[pl.ds(r0, ROWS_A)],
                        out.at[c].at[pl.ds(r0, ROWS_A)])

    @pl.when(s == 15)
    def _():
        pltpu.sync_copy(acc.at[pl.ds(15 * ROWS_A, ROWS_B)],
                        out.at[c].at[pl.ds(15 * ROWS_A, ROWS_B)])


RB = 400

def _dinv_block(dp):
    deg = dp[0, :, 0] + dp[1, :, 0] + 1.0
    return lax.rsqrt(deg)


def _mm1_body(x_ref, w_ref, o_ref):
    o_ref[...] = jnp.dot(x_ref[...], w_ref[...],
                         preferred_element_type=jnp.float32)


_mm1 = pl.pallas_call(
    _mm1_body,
    grid=(N // RB,),
    in_specs=[
        pl.BlockSpec((RB, D), lambda i: (i, 0)),
        pl.BlockSpec((D, D), lambda i: (0, 0)),
    ],
    out_specs=pl.BlockSpec((RB, D), lambda i: (i, 0)),
    out_shape=jax.ShapeDtypeStruct((N, D), jnp.float32),
)


def _scale_body(dp_ref, h_ref, o_ref):
    hs = h_ref[...] * _dinv_block(dp_ref[...])[:, None]
    o_ref[0] = hs[:, :H]
    o_ref[1] = hs[:, H:]


_scale = pl.pallas_call(
    _scale_body,
    grid=(N // RB,),
    in_specs=[
        pl.BlockSpec((NC, RB, 1), lambda i: (0, i, 0)),
        pl.BlockSpec((RB, D), lambda i: (i, 0)),
    ],
    out_specs=pl.BlockSpec((NC, RB, H), lambda i: (0, i, 0)),
    out_shape=jax.ShapeDtypeStruct((NC, N, H), jnp.float32),
)


def _mid_body(dp_ref, agg_ref, hs_ref, b_ref, w_ref, o_ref):
    dinv = _dinv_block(dp_ref[...])
    pre = jnp.concatenate(
        [agg_ref[0] + hs_ref[0], agg_ref[1] + hs_ref[1]], axis=1)
    t = jnp.maximum(pre * dinv[:, None] + b_ref[...], 0.0)
    h2 = jnp.dot(t, w_ref[...], preferred_element_type=jnp.float32)
    hs2 = h2 * dinv[:, None]
    o_ref[0] = hs2[:, :H]
    o_ref[1] = hs2[:, H:]


_mid = pl.pallas_call(
    _mid_body,
    grid=(N // RB,),
    in_specs=[
        pl.BlockSpec((NC, RB, 1), lambda i: (0, i, 0)),
        pl.BlockSpec((NC, RB, H), lambda i: (0, i, 0)),
        pl.BlockSpec((NC, RB, H), lambda i: (0, i, 0)),
        pl.BlockSpec((1, D), lambda i: (0, 0)),
        pl.BlockSpec((D, D), lambda i: (0, 0)),
    ],
    out_specs=pl.BlockSpec((NC, RB, H), lambda i: (0, i, 0)),
    out_shape=jax.ShapeDtypeStruct((NC, N, H), jnp.float32),
)


def _fin_body(dp_ref, agg_ref, hs_ref, b_ref, o_ref):
    dinv = _dinv_block(dp_ref[...])
    pre = jnp.concatenate(
        [agg_ref[0] + hs_ref[0], agg_ref[1] + hs_ref[1]], axis=1)
    o_ref[...] = pre * dinv[:, None] + b_ref[...]


_fin = pl.pallas_call(
    _fin_body,
    grid=(N // RB,),
    in_specs=[
        pl.BlockSpec((NC, RB, 1), lambda i: (0, i, 0)),
        pl.BlockSpec((NC, RB, H), lambda i: (0, i, 0)),
        pl.BlockSpec((NC, RB, H), lambda i: (0, i, 0)),
        pl.BlockSpec((1, D), lambda i: (0, 0)),
    ],
    out_specs=pl.BlockSpec((RB, D), lambda i: (i, 0)),
    out_shape=jax.ShapeDtypeStruct((N, D), jnp.float32),
)


@jax.jit
def kernel(x, edge_idx, W1, b1, W2, b2):
    zeros1 = jnp.zeros((N,), jnp.float32)
    zeros2 = jnp.zeros((N, H), jnp.float32)
    src2d = edge_idx[0].reshape(NCHUNK, CH)
    dst2d = edge_idx[1].reshape(NCHUNK, CH)
    dp = _deg_kernel(dst2d, zeros1)
    dp3 = dp.reshape(NC, N, 1)
    h1 = _mm1(x, W1)
    hs1 = _scale(dp3, h1)
    agg1 = _agg_kernel(hs1, src2d, dst2d, zeros2)
    hs2 = _mid(dp3, agg1, hs1, b1.reshape(1, D), W2)
    agg2 = _agg_kernel(hs2, src2d, dst2d, zeros2)
    return _fin(dp3, agg2, hs2, b2.reshape(1, D))


# speedup vs baseline: 1.2982x; 1.0469x over previous
import functools

import jax
import jax.numpy as jnp
from jax import lax
from jax.experimental import pallas as pl
from jax.experimental.pallas import tpu as pltpu
from jax.experimental.pallas import tpu_sc as plsc

N = 10000
D = 256
H = 128
E = 160000

NC = 2
NS = 16

CH = 128
NCHUNK = E // CH
CPT_A = 80
CPT_B = NCHUNK - 15 * CPT_A
PH = 40

CPW_D = 40
CPW_D_LAST = 1250 - 31 * CPW_D

ROWS_A = 632
ROWS_B = N - 15 * ROWS_A

_mesh = plsc.VectorSubcoreMesh(core_axis_name="c", subcore_axis_name="s")


@functools.partial(
    pl.kernel,
    out_type=jax.ShapeDtypeStruct((NC, N), jnp.float32),
    mesh=_mesh,
    scratch_types=[
        pltpu.VMEM_SHARED((N,), jnp.float32),
        pltpu.VMEM((CH,), jnp.float32),
        pltpu.VMEM((CPW_D, CH), jnp.int32),
        pltpu.SemaphoreType.DMA,
    ],
)
def _deg_kernel(dst2d, zeros1, out, accd, ones_b, dbufd, semd):
    c = lax.axis_index("c")
    s = lax.axis_index("s")
    w = c * NS + s

    @pl.when(w < 31)
    def _():
        cs = pl.multiple_of(w * CPW_D, 8)
        pltpu.make_async_copy(dst2d.at[pl.ds(cs, CPW_D)], dbufd, semd).start()

    @pl.when(w == 31)
    def _():
        pltpu.make_async_copy(dst2d.at[pl.ds(31 * CPW_D, CPW_D_LAST)],
                              dbufd.at[pl.ds(0, CPW_D_LAST)], semd).start()

    @pl.when(s == 0)
    def _():
        pltpu.sync_copy(zeros1, accd)

    for j in range(CH // 16):
        ones_b[pl.ds(j * 16, 16)] = jnp.full((16,), 1.0, dtype=jnp.float32)

    plsc.subcore_barrier()

    @pl.when(w < 31)
    def _():
        cs = pl.multiple_of(w * CPW_D, 8)
        pltpu.make_async_copy(dst2d.at[pl.ds(cs, CPW_D)], dbufd, semd).wait()

    @pl.when(w == 31)
    def _():
        pltpu.make_async_copy(dst2d.at[pl.ds(31 * CPW_D, CPW_D_LAST)],
                              dbufd.at[pl.ds(0, CPW_D_LAST)], semd).wait()

    nw = jnp.where(w < 31, CPW_D, CPW_D_LAST)

    def fire(i, _):
        pltpu.async_copy(ones_b, accd.at[dbufd.at[i]], semd, add=True)
        return ()

    lax.fori_loop(0, nw, fire, ())

    def drain(i, _):
        pltpu.make_async_copy(ones_b, accd.at[dbufd.at[i]], semd).wait()
        return ()

    lax.fori_loop(0, nw, drain, ())

    plsc.subcore_barrier()

    @pl.when(s == 0)
    def _():
        pltpu.sync_copy(accd, out.at[c])


@functools.partial(
    pl.kernel,
    out_type=jax.ShapeDtypeStruct((NC, N, H), jnp.float32),
    mesh=_mesh,
    scratch_types=[
        pltpu.VMEM_SHARED((N, H), jnp.float32),
        pltpu.VMEM((PH, CH), jnp.int32),
        pltpu.VMEM((PH, CH), jnp.int32),
        pltpu.VMEM((2, CH, H), jnp.float32),
        pltpu.SemaphoreType.DMA,
        pltpu.SemaphoreType.DMA,
    ],
)
def _agg_kernel(hs, src2d, dst2d, zeros2, out, acc, sbuf, dbuf, gbuf,
                sem0, sem1):
    c = lax.axis_index("c")
    s = lax.axis_index("s")
    r0 = pl.multiple_of(s * ROWS_A, 8)

    def gather(i, slot, sem):
        return pltpu.make_async_copy(hs.at[c].at[sbuf.at[i]],
                                     gbuf.at[slot], sem)

    def stage0():
        cs = pl.multiple_of(jnp.where(s < 15, s * CPT_A, 15 * CPT_A), 8)
        return (pltpu.make_async_copy(src2d.at[pl.ds(cs, PH)], sbuf, sem0),
                pltpu.make_async_copy(dst2d.at[pl.ds(cs, PH)], dbuf, sem1))

    for cp in stage0():
        cp.start()

    @pl.when(s < 15)
    def _():
        pltpu.sync_copy(zeros2.at[pl.ds(r0, ROWS_A)],
                        acc.at[pl.ds(r0, ROWS_A)])

    @pl.when(s == 15)
    def _():
        pltpu.sync_copy(zeros2.at[pl.ds(15 * ROWS_A, ROWS_B)],
                        acc.at[pl.ds(15 * ROWS_A, ROWS_B)])

    plsc.subcore_barrier()
    for cp in stage0():
        cp.wait()

    for p in range(2):
        if p == 0:
            nch = PH
        else:
            @pl.when(s < 15)
            def _():
                cs = pl.multiple_of(s * CPT_A + PH, 8)
                pltpu.sync_copy(src2d.at[pl.ds(cs, PH)], sbuf)
                pltpu.sync_copy(dst2d.at[pl.ds(cs, PH)], dbuf)

            @pl.when(s == 15)
            def _():
                cs = 15 * CPT_A + PH
                nlast = CPT_B - PH
                pltpu.sync_copy(src2d.at[pl.ds(cs, nlast)],
                                sbuf.at[pl.ds(0, nlast)])
                pltpu.sync_copy(dst2d.at[pl.ds(cs, nlast)],
                                dbuf.at[pl.ds(0, nlast)])

            nch = jnp.where(s < 15, PH, CPT_B - PH)

        gather(0, 0, sem0).start()

        def body(i2, _):
            a = 2 * i2
            gather(a + 1, 1, sem1).start()
            gather(a, 0, sem0).wait()
            pltpu.sync_copy(gbuf.at[0], acc.at[dbuf.at[a]], add=True)

            @pl.when(a + 2 < nch)
            def _():
                gather(a + 2, 0, sem0).start()

            gather(a + 1, 1, sem1).wait()
            pltpu.sync_copy(gbuf.at[1], acc.at[dbuf.at[a + 1]], add=True)
            return ()

        lax.fori_loop(0, nch // 2, body, ())

    plsc.subcore_barrier()

    @pl.when(s < 15)
    def _():
        pltpu.sync_copy(acc.at[pl.ds(r0, ROWS_A)],
                        out.at[c].at[pl.ds(r0, ROWS_A)])

    @pl.when(s == 15)
    def _():
        pltpu.sync_copy(acc.at[pl.ds(15 * ROWS_A, ROWS_B)],
                        out.at[c].at[pl.ds(15 * ROWS_A, ROWS_B)])


RB = 400

def _dinv_block(dp):
    deg = dp[0, :, 0] + dp[1, :, 0] + 1.0
    return lax.rsqrt(deg)


def _mm1_body(dp_ref, x_ref, w_ref, o_ref):
    h = jnp.dot(x_ref[...], w_ref[...], preferred_element_type=jnp.float32)
    hs = h * _dinv_block(dp_ref[...])[:, None]
    o_ref[0] = hs[:, :H]
    o_ref[1] = hs[:, H:]


_mm1 = pl.pallas_call(
    _mm1_body,
    grid=(N // RB,),
    in_specs=[
        pl.BlockSpec((NC, RB, 1), lambda i: (0, i, 0)),
        pl.BlockSpec((RB, D), lambda i: (i, 0)),
        pl.BlockSpec((D, D), lambda i: (0, 0)),
    ],
    out_specs=pl.BlockSpec((NC, RB, H), lambda i: (0, i, 0)),
    out_shape=jax.ShapeDtypeStruct((NC, N, H), jnp.float32),
)


def _mid_body(dp_ref, agg_ref, hs_ref, b_ref, w_ref, o_ref):
    dinv = _dinv_block(dp_ref[...])
    pre = jnp.concatenate(
        [agg_ref[0] + hs_ref[0], agg_ref[1] + hs_ref[1]], axis=1)
    t = jnp.maximum(pre * dinv[:, None] + b_ref[...], 0.0)
    h2 = jnp.dot(t, w_ref[...], preferred_element_type=jnp.float32)
    hs2 = h2 * dinv[:, None]
    o_ref[0] = hs2[:, :H]
    o_ref[1] = hs2[:, H:]


_mid = pl.pallas_call(
    _mid_body,
    grid=(N // RB,),
    in_specs=[
        pl.BlockSpec((NC, RB, 1), lambda i: (0, i, 0)),
        pl.BlockSpec((NC, RB, H), lambda i: (0, i, 0)),
        pl.BlockSpec((NC, RB, H), lambda i: (0, i, 0)),
        pl.BlockSpec((1, D), lambda i: (0, 0)),
        pl.BlockSpec((D, D), lambda i: (0, 0)),
    ],
    out_specs=pl.BlockSpec((NC, RB, H), lambda i: (0, i, 0)),
    out_shape=jax.ShapeDtypeStruct((NC, N, H), jnp.float32),
)


def _fin_body(dp_ref, agg_ref, hs_ref, b_ref, o_ref):
    dinv = _dinv_block(dp_ref[...])
    pre = jnp.concatenate(
        [agg_ref[0] + hs_ref[0], agg_ref[1] + hs_ref[1]], axis=1)
    o_ref[...] = pre * dinv[:, None] + b_ref[...]


_fin = pl.pallas_call(
    _fin_body,
    grid=(N // RB,),
    in_specs=[
        pl.BlockSpec((NC, RB, 1), lambda i: (0, i, 0)),
        pl.BlockSpec((NC, RB, H), lambda i: (0, i, 0)),
        pl.BlockSpec((NC, RB, H), lambda i: (0, i, 0)),
        pl.BlockSpec((1, D), lambda i: (0, 0)),
    ],
    out_specs=pl.BlockSpec((RB, D), lambda i: (i, 0)),
    out_shape=jax.ShapeDtypeStruct((N, D), jnp.float32),
)


@jax.jit
def kernel(x, edge_idx, W1, b1, W2, b2):
    zeros1 = jnp.zeros((N,), jnp.float32)
    zeros2 = jnp.zeros((N, H), jnp.float32)
    src2d = edge_idx[0].reshape(NCHUNK, CH)
    dst2d = edge_idx[1].reshape(NCHUNK, CH)
    dp = _deg_kernel(dst2d, zeros1)
    dp3 = dp.reshape(NC, N, 1)
    hs1 = _mm1(dp3, x, W1)
    agg1 = _agg_kernel(hs1, src2d, dst2d, zeros2)
    hs2 = _mid(dp3, agg1, hs1, b1.reshape(1, D), W2)
    agg2 = _agg_kernel(hs2, src2d, dst2d, zeros2)
    return _fin(dp3, agg2, hs2, b2.reshape(1, D))
